# baseline (speedup 1.0000x reference)
import functools

import jax
import jax.numpy as jnp
from jax import lax
from jax.experimental import pallas as pl
from jax.experimental.pallas import tpu as pltpu

B, N1, N2, DIM = 4, 256, 256, 128
HID = 128
OUT1, OUT2 = 3, 3
TQ = 8
R = TQ * N2


def _pair_kernel(qm_ref, q_ref, d_ref, dm_ref, w0q_ref, w0d_ref, b0_ref,
                 w1a_ref, b1a_ref, w1b_ref, w2a_ref, b2a_ref, w2b_ref,
                 bout_ref, out_ref, bd_ref):
    i = pl.program_id(1)

    @pl.when(i == 0)
    def _():
        bd_ref[...] = (
            jnp.dot(d_ref[0].astype(jnp.bfloat16), w0d_ref[...],
                    preferred_element_type=jnp.float32) + b0_ref[...]
        )

    aq = jnp.dot(q_ref[0].astype(jnp.bfloat16), w0q_ref[...],
                 preferred_element_type=jnp.float32)

    aq_exp = jnp.reshape(
        jnp.broadcast_to(aq[:, None, :], (TQ, N2, HID)), (R, HID))
    bd_exp = jnp.reshape(
        jnp.broadcast_to(bd_ref[...][None, :, :], (TQ, N2, HID)), (R, HID))
    enc = jnp.maximum(aq_exp + bd_exp, 0.0).astype(jnp.bfloat16)

    h1 = jnp.maximum(
        jnp.dot(enc, w1a_ref[...], preferred_element_type=jnp.float32)
        + b1a_ref[...], 0.0).astype(jnp.bfloat16)
    s1 = jnp.dot(h1, w1b_ref[...], preferred_element_type=jnp.float32)
    h2 = jnp.maximum(
        jnp.dot(enc, w2a_ref[...], preferred_element_type=jnp.float32)
        + b2a_ref[...], 0.0).astype(jnp.bfloat16)
    s2 = jnp.dot(h2, w2b_ref[...], preferred_element_type=jnp.float32)
    s = jnp.concatenate([s1, s2], axis=1) + bout_ref[...]

    ridx = lax.broadcasted_iota(jnp.int32, (R, 1), 0)
    t = ridx // N2
    qmask = jnp.zeros((R, 1), jnp.float32)
    for tt in range(TQ):
        qs = qm_ref[0, 0, 0, tt].astype(jnp.float32)
        qmask = jnp.where(t == tt, qs, qmask)
    dmask = jnp.concatenate([dm_ref[0]] * TQ, axis=0)
    out_ref[...] = s * (qmask * dmask)


@jax.jit
def kernel(query, doc, query_mask, doc_mask, W0, b0, Wp1a, bp1a, Wp1b, bp1b,
           Wp2a, bp2a, Wp2b, bp2b):
    w0q = W0[:DIM].astype(jnp.bfloat16)
    w0d = W0[DIM:].astype(jnp.bfloat16)
    w1a = Wp1a.astype(jnp.bfloat16)
    w2a = Wp2a.astype(jnp.bfloat16)
    w1b = Wp1b.astype(jnp.bfloat16)
    w2b = Wp2b.astype(jnp.bfloat16)
    b0r = b0.reshape(1, HID)
    b1ar = bp1a.reshape(1, HID)
    b2ar = bp2a.reshape(1, HID)
    bout = jnp.concatenate([bp1b, bp2b]).reshape(1, OUT1 + OUT2)
    qm = query_mask.astype(jnp.int32).reshape(B, N1 // TQ, 1, TQ)
    dm = doc_mask.astype(jnp.float32).reshape(B, N2, 1)

    grid = (B, N1 // TQ)
    rep = lambda b, i: (0, 0)

    out = pl.pallas_call(
        _pair_kernel,
        grid=grid,
        in_specs=[
            pl.BlockSpec((1, 1, 1, TQ), lambda b, i: (b, i, 0, 0),
                         memory_space=pltpu.SMEM),
            pl.BlockSpec((1, TQ, DIM), lambda b, i: (b, i, 0)),
            pl.BlockSpec((1, N2, DIM), lambda b, i: (b, 0, 0)),
            pl.BlockSpec((1, N2, 1), lambda b, i: (b, 0, 0)),
            pl.BlockSpec((DIM, HID), rep),
            pl.BlockSpec((DIM, HID), rep),
            pl.BlockSpec((1, HID), rep),
            pl.BlockSpec((HID, HID), rep),
            pl.BlockSpec((1, HID), rep),
            pl.BlockSpec((HID, OUT1), rep),
            pl.BlockSpec((HID, HID), rep),
            pl.BlockSpec((1, HID), rep),
            pl.BlockSpec((HID, OUT2), rep),
            pl.BlockSpec((1, OUT1 + OUT2), rep),
        ],
        out_specs=pl.BlockSpec((R, OUT1 + OUT2),
                               lambda b, i: (b * (N1 // TQ) + i, 0)),
        out_shape=jax.ShapeDtypeStruct((B * N1 * N2, OUT1 + OUT2),
                                       jnp.float32),
        scratch_shapes=[pltpu.VMEM((N2, HID), jnp.float32)],
    )(qm, query, doc, dm, w0q, w0d, b0r, w1a, b1ar, w1b, w2a, b2ar,
      w2b, bout)

    scores1 = out[:, :OUT1].reshape(B, N1, N2, OUT1)
    scores2 = out[:, OUT1:].reshape(B, N1, N2, OUT2)
    return (scores1, scores2)


# speedup vs baseline: 3.3341x; 3.3341x over previous
import jax
import jax.numpy as jnp
from jax import lax
from jax.experimental import pallas as pl
from jax.experimental.pallas import tpu as pltpu

B, N1, N2, DIM = 4, 256, 256, 128
HID = 128
OUT1, OUT2 = 3, 3
NOUT = OUT1 + OUT2
TQ = 8
R = TQ * N2


def _pair_kernel(qm_ref, q_ref, d_ref, dm_ref, w0q_ref, w0d_ref, b0_ref,
                 w1a_ref, b1a_ref, w1bt_ref, w2a_ref, b2a_ref, w2bt_ref,
                 bout_ref, out_ref, bd_ref, dmt_ref):
    i = pl.program_id(1)

    @pl.when(i == 0)
    def _():
        bd_ref[...] = (
            jnp.dot(d_ref[0].astype(jnp.bfloat16), w0d_ref[...],
                    preferred_element_type=jnp.float32) + b0_ref[...]
        )
        dmt_ref[...] = jnp.concatenate([dm_ref[0]] * TQ, axis=1)

    aq = jnp.dot(q_ref[0].astype(jnp.bfloat16), w0q_ref[...],
                 preferred_element_type=jnp.float32)

    aq_exp = jnp.reshape(
        jnp.broadcast_to(aq[:, None, :], (TQ, N2, HID)), (R, HID))
    bd_exp = jnp.reshape(
        jnp.broadcast_to(bd_ref[...][None, :, :], (TQ, N2, HID)), (R, HID))
    enc = jnp.maximum(aq_exp + bd_exp, 0.0).astype(jnp.bfloat16)

    h1 = jnp.maximum(
        jnp.dot(enc, w1a_ref[...], preferred_element_type=jnp.float32)
        + b1a_ref[...], 0.0).astype(jnp.bfloat16)
    h2 = jnp.maximum(
        jnp.dot(enc, w2a_ref[...], preferred_element_type=jnp.float32)
        + b2a_ref[...], 0.0).astype(jnp.bfloat16)
    s1t = lax.dot_general(w1bt_ref[...], h1, (((1,), (1,)), ((), ())),
                          preferred_element_type=jnp.float32)
    s2t = lax.dot_general(w2bt_ref[...], h2, (((1,), (1,)), ((), ())),
                          preferred_element_type=jnp.float32)
    st = jnp.concatenate([s1t, s2t], axis=0) + bout_ref[...]

    lidx = lax.broadcasted_iota(jnp.int32, (1, R), 1) // N2
    qmask = jnp.zeros((1, R), jnp.float32)
    for tt in range(TQ):
        qs = qm_ref[0, 0, 0, tt].astype(jnp.float32)
        qmask = jnp.where(lidx == tt, qs, qmask)
    out_ref[...] = st * (qmask * dmt_ref[...])


@jax.jit
def kernel(query, doc, query_mask, doc_mask, W0, b0, Wp1a, bp1a, Wp1b, bp1b,
           Wp2a, bp2a, Wp2b, bp2b):
    w0q = W0[:DIM].astype(jnp.bfloat16)
    w0d = W0[DIM:].astype(jnp.bfloat16)
    w1a = Wp1a.astype(jnp.bfloat16)
    w2a = Wp2a.astype(jnp.bfloat16)
    w1bt = Wp1b.T.astype(jnp.bfloat16)
    w2bt = Wp2b.T.astype(jnp.bfloat16)
    b0r = b0.reshape(1, HID)
    b1ar = bp1a.reshape(1, HID)
    b2ar = bp2a.reshape(1, HID)
    bout = jnp.concatenate([bp1b, bp2b]).reshape(NOUT, 1)
    qm = query_mask.astype(jnp.int32).reshape(B, N1 // TQ, 1, TQ)
    dm = doc_mask.astype(jnp.float32).reshape(B, 1, N2)

    grid = (B, N1 // TQ)
    rep = lambda b, i: (0, 0)

    out = pl.pallas_call(
        _pair_kernel,
        grid=grid,
        in_specs=[
            pl.BlockSpec((1, 1, 1, TQ), lambda b, i: (b, i, 0, 0),
                         memory_space=pltpu.SMEM),
            pl.BlockSpec((1, TQ, DIM), lambda b, i: (b, i, 0)),
            pl.BlockSpec((1, N2, DIM), lambda b, i: (b, 0, 0)),
            pl.BlockSpec((1, 1, N2), lambda b, i: (b, 0, 0)),
            pl.BlockSpec((DIM, HID), rep),
            pl.BlockSpec((DIM, HID), rep),
            pl.BlockSpec((1, HID), rep),
            pl.BlockSpec((HID, HID), rep),
            pl.BlockSpec((1, HID), rep),
            pl.BlockSpec((OUT1, HID), rep),
            pl.BlockSpec((HID, HID), rep),
            pl.BlockSpec((1, HID), rep),
            pl.BlockSpec((OUT2, HID), rep),
            pl.BlockSpec((NOUT, 1), rep),
        ],
        out_specs=pl.BlockSpec((NOUT, R),
                               lambda b, i: (0, b * (N1 // TQ) + i)),
        out_shape=jax.ShapeDtypeStruct((NOUT, B * N1 * N2), jnp.float32),
        scratch_shapes=[pltpu.VMEM((N2, HID), jnp.float32),
                        pltpu.VMEM((1, R), jnp.float32)],
    )(qm, query, doc, dm, w0q, w0d, b0r, w1a, b1ar, w1bt, w2a, b2ar,
      w2bt, bout)

    st = out.T
    scores1 = st[:, :OUT1].reshape(B, N1, N2, OUT1)
    scores2 = st[:, OUT1:].reshape(B, N1, N2, OUT2)
    return (scores1, scores2)


# speedup vs baseline: 4.0551x; 1.2162x over previous
import jax
import jax.numpy as jnp
from jax import lax
from jax.experimental import pallas as pl
from jax.experimental.pallas import tpu as pltpu

B, N1, N2, DIM = 4, 256, 256, 128
HID = 128
OUT1, OUT2 = 3, 3
NOUT = OUT1 + OUT2
TQ = 16
SUB = 8
NSUB = TQ // SUB
R = TQ * N2
RH = SUB * N2


def _pair_kernel(qm_ref, q_ref, d_ref, dm_ref, w0q_ref, w0d_ref, b0_ref,
                 w1a_ref, b1a_ref, w1bt_ref, w2a_ref, b2a_ref, w2bt_ref,
                 bout_ref, out_ref, aq_ref, bd_ref, dmt_ref):
    i = pl.program_id(1)

    @pl.when(i == 0)
    def _():
        aq_ref[...] = jnp.dot(q_ref[0].astype(jnp.bfloat16), w0q_ref[...],
                              preferred_element_type=jnp.float32)
        bd_ref[...] = (
            jnp.dot(d_ref[0].astype(jnp.bfloat16), w0d_ref[...],
                    preferred_element_type=jnp.float32) + b0_ref[...]
        )
        dmt_ref[...] = jnp.concatenate([dm_ref[0]] * TQ, axis=1)

    base = i * TQ
    bd = bd_ref[...]
    sts = []
    for h in range(NSUB):
        aqr = aq_ref[pl.ds(base + h * SUB, SUB), :]
        enc = jnp.reshape(
            jnp.maximum(aqr[:, None, :] + bd[None, :, :], 0.0),
            (RH, HID)).astype(jnp.bfloat16)
        h1 = jnp.maximum(
            jnp.dot(enc, w1a_ref[...], preferred_element_type=jnp.float32)
            + b1a_ref[...], 0.0).astype(jnp.bfloat16)
        h2 = jnp.maximum(
            jnp.dot(enc, w2a_ref[...], preferred_element_type=jnp.float32)
            + b2a_ref[...], 0.0).astype(jnp.bfloat16)
        s1t = lax.dot_general(w1bt_ref[...], h1, (((1,), (1,)), ((), ())),
                              preferred_element_type=jnp.float32)
        s2t = lax.dot_general(w2bt_ref[...], h2, (((1,), (1,)), ((), ())),
                              preferred_element_type=jnp.float32)
        sts.append(jnp.concatenate([s1t, s2t], axis=0))
    st = jnp.concatenate(sts, axis=1) + bout_ref[...]

    lidx = lax.broadcasted_iota(jnp.int32, (1, R), 1) // N2
    qmask = jnp.zeros((1, R), jnp.float32)
    for tt in range(TQ):
        qs = qm_ref[0, 0, 0, tt].astype(jnp.float32)
        qmask = jnp.where(lidx == tt, qs, qmask)
    out_ref[...] = st * (qmask * dmt_ref[...])


@jax.jit
def kernel(query, doc, query_mask, doc_mask, W0, b0, Wp1a, bp1a, Wp1b, bp1b,
           Wp2a, bp2a, Wp2b, bp2b):
    w0q = W0[:DIM].astype(jnp.bfloat16)
    w0d = W0[DIM:].astype(jnp.bfloat16)
    w1a = Wp1a.astype(jnp.bfloat16)
    w2a = Wp2a.astype(jnp.bfloat16)
    w1bt = Wp1b.T.astype(jnp.bfloat16)
    w2bt = Wp2b.T.astype(jnp.bfloat16)
    b0r = b0.reshape(1, HID)
    b1ar = bp1a.reshape(1, HID)
    b2ar = bp2a.reshape(1, HID)
    bout = jnp.concatenate([bp1b, bp2b]).reshape(NOUT, 1)
    qm = query_mask.astype(jnp.int32).reshape(B, N1 // TQ, 1, TQ)
    dm = doc_mask.astype(jnp.float32).reshape(B, 1, N2)

    grid = (B, N1 // TQ)
    rep = lambda b, i: (0, 0)

    out = pl.pallas_call(
        _pair_kernel,
        grid=grid,
        in_specs=[
            pl.BlockSpec((1, 1, 1, TQ), lambda b, i: (b, i, 0, 0),
                         memory_space=pltpu.SMEM),
            pl.BlockSpec((1, N1, DIM), lambda b, i: (b, 0, 0)),
            pl.BlockSpec((1, N2, DIM), lambda b, i: (b, 0, 0)),
            pl.BlockSpec((1, 1, N2), lambda b, i: (b, 0, 0)),
            pl.BlockSpec((DIM, HID), rep),
            pl.BlockSpec((DIM, HID), rep),
            pl.BlockSpec((1, HID), rep),
            pl.BlockSpec((HID, HID), rep),
            pl.BlockSpec((1, HID), rep),
            pl.BlockSpec((OUT1, HID), rep),
            pl.BlockSpec((HID, HID), rep),
            pl.BlockSpec((1, HID), rep),
            pl.BlockSpec((OUT2, HID), rep),
            pl.BlockSpec((NOUT, 1), rep),
        ],
        out_specs=pl.BlockSpec((NOUT, R),
                               lambda b, i: (0, b * (N1 // TQ) + i)),
        out_shape=jax.ShapeDtypeStruct((NOUT, B * N1 * N2), jnp.float32),
        scratch_shapes=[pltpu.VMEM((N1, HID), jnp.float32),
                        pltpu.VMEM((N2, HID), jnp.float32),
                        pltpu.VMEM((1, R), jnp.float32)],
    )(qm, query, doc, dm, w0q, w0d, b0r, w1a, b1ar, w1bt, w2a, b2ar,
      w2bt, bout)

    st = out.T
    scores1 = st[:, :OUT1].reshape(B, N1, N2, OUT1)
    scores2 = st[:, OUT1:].reshape(B, N1, N2, OUT2)
    return (scores1, scores2)


# speedup vs baseline: 4.2036x; 1.0366x over previous
import jax
import jax.numpy as jnp
from jax import lax
from jax.experimental import pallas as pl
from jax.experimental.pallas import tpu as pltpu

B, N1, N2, DIM = 4, 256, 256, 128
HID = 128
OUT1, OUT2 = 3, 3
NOUT = OUT1 + OUT2
TQ = 16
SUB = 8
NSUB = TQ // SUB
R = TQ * N2
RH = SUB * N2


def _pair_kernel(qm_ref, q_ref, d_ref, dm_ref, w0q_ref, w0d_ref, b0_ref,
                 w12a_ref, b12a_ref, wbt_ref, bout_ref, out_ref,
                 aq_ref, bd_ref, dmt_ref):
    i = pl.program_id(1)

    @pl.when(i == 0)
    def _():
        aq_ref[...] = jnp.dot(q_ref[0].astype(jnp.bfloat16), w0q_ref[...],
                              preferred_element_type=jnp.float32)
        bd_ref[...] = (
            jnp.dot(d_ref[0].astype(jnp.bfloat16), w0d_ref[...],
                    preferred_element_type=jnp.float32) + b0_ref[...]
        )
        dmt_ref[...] = jnp.concatenate([dm_ref[0]] * TQ, axis=1)

    base = i * TQ
    bd = bd_ref[...]
    sts = []
    for h in range(NSUB):
        aqr = aq_ref[pl.ds(base + h * SUB, SUB), :]
        enc = jnp.reshape(
            jnp.maximum(aqr[:, None, :] + bd[None, :, :], 0.0),
            (RH, HID)).astype(jnp.bfloat16)
        h12 = jnp.maximum(
            jnp.dot(enc, w12a_ref[...], preferred_element_type=jnp.float32)
            + b12a_ref[...], 0.0).astype(jnp.bfloat16)
        sts.append(
            lax.dot_general(wbt_ref[...], h12, (((1,), (1,)), ((), ())),
                            preferred_element_type=jnp.float32))
    st = jnp.concatenate(sts, axis=1) + bout_ref[...]

    lidx = lax.broadcasted_iota(jnp.int32, (1, R), 1) // N2
    qmask = jnp.zeros((1, R), jnp.float32)
    for tt in range(TQ):
        qs = qm_ref[0, 0, 0, tt].astype(jnp.float32)
        qmask = jnp.where(lidx == tt, qs, qmask)
    out_ref[...] = st * (qmask * dmt_ref[...])


@jax.jit
def kernel(query, doc, query_mask, doc_mask, W0, b0, Wp1a, bp1a, Wp1b, bp1b,
           Wp2a, bp2a, Wp2b, bp2b):
    w0q = W0[:DIM].astype(jnp.bfloat16)
    w0d = W0[DIM:].astype(jnp.bfloat16)
    w12a = jnp.concatenate([Wp1a, Wp2a], axis=1).astype(jnp.bfloat16)
    wbt = jnp.concatenate([
        jnp.concatenate([Wp1b.T, jnp.zeros((OUT1, HID), Wp1b.dtype)], axis=1),
        jnp.concatenate([jnp.zeros((OUT2, HID), Wp2b.dtype), Wp2b.T], axis=1),
    ], axis=0).astype(jnp.bfloat16)
    b0r = b0.reshape(1, HID)
    b12ar = jnp.concatenate([bp1a, bp2a]).reshape(1, 2 * HID)
    bout = jnp.concatenate([bp1b, bp2b]).reshape(NOUT, 1)
    qm = query_mask.astype(jnp.int32).reshape(B, N1 // TQ, 1, TQ)
    dm = doc_mask.astype(jnp.float32).reshape(B, 1, N2)

    grid = (B, N1 // TQ)
    rep = lambda b, i: (0, 0)

    out = pl.pallas_call(
        _pair_kernel,
        grid=grid,
        in_specs=[
            pl.BlockSpec((1, 1, 1, TQ), lambda b, i: (b, i, 0, 0),
                         memory_space=pltpu.SMEM),
            pl.BlockSpec((1, N1, DIM), lambda b, i: (b, 0, 0)),
            pl.BlockSpec((1, N2, DIM), lambda b, i: (b, 0, 0)),
            pl.BlockSpec((1, 1, N2), lambda b, i: (b, 0, 0)),
            pl.BlockSpec((DIM, HID), rep),
            pl.BlockSpec((DIM, HID), rep),
            pl.BlockSpec((1, HID), rep),
            pl.BlockSpec((HID, 2 * HID), rep),
            pl.BlockSpec((1, 2 * HID), rep),
            pl.BlockSpec((NOUT, 2 * HID), rep),
            pl.BlockSpec((NOUT, 1), rep),
        ],
        out_specs=pl.BlockSpec((NOUT, R),
                               lambda b, i: (0, b * (N1 // TQ) + i)),
        out_shape=jax.ShapeDtypeStruct((NOUT, B * N1 * N2), jnp.float32),
        scratch_shapes=[pltpu.VMEM((N1, HID), jnp.float32),
                        pltpu.VMEM((N2, HID), jnp.float32),
                        pltpu.VMEM((1, R), jnp.float32)],
    )(qm, query, doc, dm, w0q, w0d, b0r, w12a, b12ar, wbt, bout)

    st = out.T
    scores1 = st[:, :OUT1].reshape(B, N1, N2, OUT1)
    scores2 = st[:, OUT1:].reshape(B, N1, N2, OUT2)
    return (scores1, scores2)
